# baseline (speedup 1.0000x reference)
import functools

import jax
import jax.numpy as jnp
from jax import lax
from jax.experimental import pallas as pl
from jax.experimental.pallas import tpu as pltpu
from jax.experimental.pallas import tpu_sc as plsc

NC, NS, LANES = 2, 16, 16
NW = NC * NS
K = 128
CH = 80
EPW = CH * K
TN = 400


def _mix_body(comp_ref, basis_ref, w_ref):
    r_cnt, b_cnt = comp_ref.shape
    for r in range(r_cnt):
        w = comp_ref[r, 0] * basis_ref[0]
        for b in range(1, b_cnt):
            w = w + comp_ref[r, b] * basis_ref[b]
        w_ref[r] = w


def _mix(comp, basis):
    r_cnt, _ = comp.shape
    _, d, _ = basis.shape
    return pl.pallas_call(
        _mix_body,
        out_shape=jax.ShapeDtypeStruct((r_cnt, d, d), jnp.float32),
    )(comp, basis)


def _project_body(w_ref, h_ref, out_ref):
    out_ref[0] = jnp.dot(h_ref[...], w_ref[0], preferred_element_type=jnp.float32)


def _project(h, w):
    n, d = h.shape
    r_cnt = w.shape[0]
    return pl.pallas_call(
        _project_body,
        grid=(n // TN, r_cnt),
        in_specs=[
            pl.BlockSpec((1, d, d), lambda n_i, r: (r, 0, 0)),
            pl.BlockSpec((TN, d), lambda n_i, r: (n_i, 0)),
        ],
        out_specs=pl.BlockSpec((1, TN, d), lambda n_i, r: (r, n_i, 0)),
        out_shape=jax.ShapeDtypeStruct((r_cnt, n, d), jnp.float32),
    )(w, h)


def _deg_reduce_body(degp_ref, out_ref):
    out_ref[...] = (degp_ref[0, :, 0] + degp_ref[1, :, 0])[:, None]


def _deg_reduce(degp):
    nc, n_acc, d = degp.shape
    return pl.pallas_call(
        _deg_reduce_body,
        grid=(n_acc // K,),
        in_specs=[pl.BlockSpec((nc, K, d), lambda i: (0, i, 0))],
        out_specs=pl.BlockSpec((K, 1), lambda i: (i, 0)),
        out_shape=jax.ShapeDtypeStruct((n_acc, 1), jnp.float32),
    )(degp)


def _combine_body(accp_ref, degp_ref, h_ref, wself_ref, b_ref, out_ref):
    a = accp_ref[0] + accp_ref[1]
    deg = degp_ref[...]
    rdeg = 1.0 / jnp.maximum(deg, 1.0)
    out = a * rdeg + jnp.dot(h_ref[...], wself_ref[...],
                             preferred_element_type=jnp.float32) + b_ref[0]
    out_ref[...] = jnp.maximum(out, 0.0)


def _combine(accp, degp, h, w_self, bias):
    n, d = h.shape
    return pl.pallas_call(
        _combine_body,
        grid=(n // TN,),
        in_specs=[
            pl.BlockSpec((NC, TN, d), lambda i: (0, i, 0)),
            pl.BlockSpec((TN, 1), lambda i: (i, 0)),
            pl.BlockSpec((TN, d), lambda i: (i, 0)),
            pl.BlockSpec((d, d), lambda i: (0, 0)),
            pl.BlockSpec((1, d), lambda i: (0, 0)),
        ],
        out_specs=pl.BlockSpec((TN, d), lambda i: (i, 0)),
        out_shape=jax.ShapeDtypeStruct((n, d), jnp.float32),
    )(accp, degp, h, w_self, bias.reshape(1, d))


def _gidx_body(src_ref, typ_ref, out_ref, *, n_nodes):
    out_ref[...] = typ_ref[...] * n_nodes + src_ref[...]


def _gidx(src2, typ2, n_nodes):
    rows, k = src2.shape
    tr = NW * 2
    return pl.pallas_call(
        functools.partial(_gidx_body, n_nodes=n_nodes),
        grid=(rows // tr,),
        in_specs=[pl.BlockSpec((tr, k), lambda i: (i, 0)),
                  pl.BlockSpec((tr, k), lambda i: (i, 0))],
        out_specs=pl.BlockSpec((tr, k), lambda i: (i, 0)),
        out_shape=jax.ShapeDtypeStruct((rows, k), jnp.int32),
    )(src2, typ2)


K2 = 64
NBUF = 4
STAGE = 32
CPT = NW * EPW // (NS * K2)


@functools.cache
def _make_aggregate(n_nodes, d, n_acc):
    mesh = plsc.VectorSubcoreMesh(core_axis_name="c", subcore_axis_name="s",
                                  num_cores=NC, num_subcores=NS)
    out_type = jax.ShapeDtypeStruct((NC, n_acc, d), jnp.float32)
    scratch = [
        pltpu.VMEM_SHARED((n_acc, d), jnp.float32),
        pltpu.VMEM((STAGE, K2), jnp.int32),
        pltpu.VMEM((STAGE, K2), jnp.int32),
        [pltpu.VMEM((K2, d), jnp.float32) for _ in range(NBUF)],
        [pltpu.SemaphoreType.DMA for _ in range(NBUF)],
        [pltpu.SemaphoreType.DMA for _ in range(NBUF)],
    ]
    rows_per_tile = n_acc // NS

    def body(gidx2, dst2, table, accout, acc_sh, gidxv, dstv, rows, gsem, ssem):
        c = lax.axis_index("c")
        s = lax.axis_index("s")

        zeros16 = jnp.zeros((LANES,), jnp.float32)

        def init_row(i, carry):
            for k in range(d // LANES):
                rows[0][i, pl.ds(k * LANES, LANES)] = zeros16
            return carry
        lax.fori_loop(0, K2, init_row, 0)

        for j in range(rows_per_tile // K2):
            base = s * rows_per_tile + j * K2
            pltpu.sync_copy(rows[0].at[pl.ds(0, K2)], acc_sh.at[pl.ds(base, K2)])
        plsc.subcore_barrier()

        def gather(j, b):
            return pltpu.async_copy(table.at[gidxv.at[j]], rows[b], gsem[b])

        def gwait(j, b):
            pltpu.make_async_copy(table.at[gidxv.at[j]], rows[b], gsem[b]).wait()

        def scatter(j, b):
            pltpu.async_copy(rows[b], acc_sh.at[dstv.at[j]], ssem[b], add=True)

        def swait(j, b):
            pltpu.make_async_copy(rows[b], acc_sh.at[dstv.at[j]], ssem[b]).wait()

        def run_stage(base_row):
            pltpu.sync_copy(gidx2.at[pl.ds(base_row, STAGE)], gidxv)
            pltpu.sync_copy(dst2.at[pl.ds(base_row, STAGE)], dstv)

            gather(0, 0)
            gather(1, 1)

            def it(i, carry):
                for b in range(NBUF):
                    j = i * NBUF + b
                    gwait(j, b)
                    scatter(j, b)
                    jn = j + 2
                    bn = (b + 2) % NBUF

                    @pl.when(jn < STAGE)
                    def _():
                        @pl.when(jn >= NBUF)
                        def _():
                            swait(jn - NBUF, bn)
                        gather(jn, bn)
                return carry
            lax.fori_loop(0, STAGE // NBUF, it, 0)

            for b in range(NBUF):
                swait(STAGE - NBUF + b, b)

        @pl.when(c == 0)
        def _():
            for h in range(CPT // STAGE):
                run_stage(s * CPT + h * STAGE)

        plsc.subcore_barrier()
        for j in range(rows_per_tile // K2):
            base = s * rows_per_tile + j * K2
            pltpu.sync_copy(acc_sh.at[pl.ds(base, K2)], accout.at[c, pl.ds(base, K2)])

    return pl.kernel(body, out_type=out_type, mesh=mesh, scratch_types=scratch)


@functools.cache
def _make_deg(n_acc, d):
    mesh = plsc.VectorSubcoreMesh(core_axis_name="c", subcore_axis_name="s",
                                  num_cores=NC, num_subcores=NS)
    out_type = jax.ShapeDtypeStruct((NC, n_acc, d), jnp.float32)
    scratch = [
        pltpu.VMEM_SHARED((n_acc, d), jnp.float32),
        pltpu.VMEM((CH, K), jnp.int32),
        pltpu.VMEM((K, d), jnp.float32),
        pltpu.SemaphoreType.DMA,
    ]
    rows_per_tile = n_acc // NS

    def body(dst2, zrows, degout, deg_sh, dstv, onesv, sem):
        c = lax.axis_index("c")
        s = lax.axis_index("s")
        wid = c * NS + s
        ones16 = jnp.ones((LANES,), jnp.float32)

        def init_row(i, carry):
            for k in range(d // LANES):
                onesv[i, pl.ds(k * LANES, LANES)] = ones16
            return carry
        lax.fori_loop(0, K, init_row, 0)

        for j in range(rows_per_tile // K):
            base = s * rows_per_tile + j * K
            pltpu.sync_copy(zrows, deg_sh.at[pl.ds(base, K)])
        plsc.subcore_barrier()

        pltpu.sync_copy(dst2.at[pl.ds(wid * CH, CH)], dstv)

        def chunk(j, carry):
            pltpu.async_copy(onesv, deg_sh.at[dstv.at[j]], sem, add=True)
            return carry
        lax.fori_loop(0, CH, chunk, 0)

        def drain(j, carry):
            pltpu.make_async_copy(onesv, deg_sh.at[dstv.at[j]], sem).wait()
            return carry
        lax.fori_loop(0, CH, drain, 0)

        plsc.subcore_barrier()
        for j in range(rows_per_tile // K):
            base = s * rows_per_tile + j * K
            pltpu.sync_copy(deg_sh.at[pl.ds(base, K)], degout.at[c, pl.ds(base, K)])

    return pl.kernel(body, out_type=out_type, mesh=mesh, scratch_types=scratch)


@functools.cache
def _make_gather(n_q, d):
    mesh = plsc.VectorSubcoreMesh(core_axis_name="c", subcore_axis_name="s",
                                  num_cores=NC, num_subcores=NS)
    rows_pw = n_q // NW
    out_type = [jax.ShapeDtypeStruct((n_q, d), jnp.float32),
                jax.ShapeDtypeStruct((n_q, d), jnp.float32)]
    scratch = [
        pltpu.VMEM((rows_pw,), jnp.int32),
        pltpu.VMEM((rows_pw, d), jnp.float32),
        pltpu.SemaphoreType.DMA,
    ]

    def body(h1_hbm, h2_hbm, ids_hbm, o1, o2, idxv, rowsv, sem):
        c = lax.axis_index("c")
        s = lax.axis_index("s")
        base = (c * NS + s) * rows_pw
        pltpu.sync_copy(ids_hbm.at[pl.ds(base, rows_pw)], idxv)
        pltpu.async_copy(h1_hbm.at[idxv], rowsv, sem).wait()
        pltpu.sync_copy(rowsv, o1.at[pl.ds(base, rows_pw)])
        pltpu.async_copy(h2_hbm.at[idxv], rowsv, sem).wait()
        pltpu.sync_copy(rowsv, o2.at[pl.ds(base, rows_pw)])

    return pl.kernel(body, out_type=out_type, mesh=mesh, scratch_types=scratch)


def kernel(x, edge_index, edge_type, node_ids, W1_basis, W1_comp, W1_self, b1,
           W2_basis, W2_comp, W2_self, b2):
    n, d = x.shape
    e = edge_type.shape[0]
    n_q = node_ids.shape[0]
    n_acc = ((n + 1 + NS * K - 1) // (NS * K)) * (NS * K)

    e_pad = NW * EPW
    pad = e_pad - e
    src2 = jnp.concatenate([edge_index[0], jnp.zeros((pad,), jnp.int32)]).reshape(NW * CH, K)
    dst1 = jnp.concatenate([edge_index[1], jnp.full((pad,), n, jnp.int32)])
    dst2 = dst1.reshape(NW * CH, K)
    typ2 = jnp.concatenate([edge_type, jnp.zeros((pad,), jnp.int32)]).reshape(NW * CH, K)

    agg = _make_aggregate(n, d, n_acc)
    deg_k = _make_deg(n_acc, d)
    gather2 = _make_gather(n_q, d)

    zrows = jnp.zeros((K, d), jnp.float32)
    degp = _deg_reduce(deg_k(dst2, zrows))
    gidx2 = _gidx(src2, typ2, n)

    gidx2r = gidx2.reshape(-1, K2)
    dst2r = dst2.reshape(-1, K2)

    w1 = _mix(W1_comp, W1_basis)
    hrel1 = _project(x, w1).reshape(-1, d)
    accp1 = agg(gidx2r, dst2r, hrel1)
    h1 = _combine(accp1, degp, x, W1_self, b1)

    w2 = _mix(W2_comp, W2_basis)
    hrel2 = _project(h1, w2).reshape(-1, d)
    accp2 = agg(gidx2r, dst2r, hrel2)
    h2 = _combine(accp2, degp, h1, W2_self, b2)

    g1, g2 = gather2(h1, h2, node_ids)
    return jnp.concatenate([g1, g2], axis=-1)


# speedup vs baseline: 1.3102x; 1.3102x over previous
import functools

import jax
import jax.numpy as jnp
from jax import lax
from jax.experimental import pallas as pl
from jax.experimental.pallas import tpu as pltpu
from jax.experimental.pallas import tpu_sc as plsc

NC, NS, LANES = 2, 16, 16
NW = NC * NS
K = 128
CH = 80
EPW = CH * K
TN = 400


def _mix_body(comp_ref, basis_ref, w_ref):
    r_cnt, b_cnt = comp_ref.shape
    for r in range(r_cnt):
        w = comp_ref[r, 0] * basis_ref[0]
        for b in range(1, b_cnt):
            w = w + comp_ref[r, b] * basis_ref[b]
        w_ref[r] = w


def _mix(comp, basis):
    r_cnt, _ = comp.shape
    _, d, _ = basis.shape
    return pl.pallas_call(
        _mix_body,
        out_shape=jax.ShapeDtypeStruct((r_cnt, d, d), jnp.float32),
    )(comp, basis)


def _project_body(w_ref, h_ref, out_ref):
    out_ref[0] = jnp.dot(h_ref[...], w_ref[0], preferred_element_type=jnp.float32)


def _project(h, w):
    n, d = h.shape
    r_cnt = w.shape[0]
    return pl.pallas_call(
        _project_body,
        grid=(n // TN, r_cnt),
        in_specs=[
            pl.BlockSpec((1, d, d), lambda n_i, r: (r, 0, 0)),
            pl.BlockSpec((TN, d), lambda n_i, r: (n_i, 0)),
        ],
        out_specs=pl.BlockSpec((1, TN, d), lambda n_i, r: (r, n_i, 0)),
        out_shape=jax.ShapeDtypeStruct((r_cnt, n, d), jnp.float32),
    )(w, h)


def _deg_reduce_body(degp_ref, out_ref):
    out_ref[...] = (degp_ref[0, :, 0] + degp_ref[1, :, 0])[:, None]


def _deg_reduce(degp):
    nc, n_acc, d = degp.shape
    return pl.pallas_call(
        _deg_reduce_body,
        grid=(n_acc // K,),
        in_specs=[pl.BlockSpec((nc, K, d), lambda i: (0, i, 0))],
        out_specs=pl.BlockSpec((K, 1), lambda i: (i, 0)),
        out_shape=jax.ShapeDtypeStruct((n_acc, 1), jnp.float32),
    )(degp)


def _combine_body(accp_ref, degp_ref, h_ref, wself_ref, b_ref, out_ref):
    a = accp_ref[0] + accp_ref[1]
    deg = degp_ref[...]
    rdeg = 1.0 / jnp.maximum(deg, 1.0)
    out = a * rdeg + jnp.dot(h_ref[...], wself_ref[...],
                             preferred_element_type=jnp.float32) + b_ref[0]
    out_ref[...] = jnp.maximum(out, 0.0)


def _combine(accp, degp, h, w_self, bias):
    n, d = h.shape
    return pl.pallas_call(
        _combine_body,
        grid=(n // TN,),
        in_specs=[
            pl.BlockSpec((NC, TN, d), lambda i: (0, i, 0)),
            pl.BlockSpec((TN, 1), lambda i: (i, 0)),
            pl.BlockSpec((TN, d), lambda i: (i, 0)),
            pl.BlockSpec((d, d), lambda i: (0, 0)),
            pl.BlockSpec((1, d), lambda i: (0, 0)),
        ],
        out_specs=pl.BlockSpec((TN, d), lambda i: (i, 0)),
        out_shape=jax.ShapeDtypeStruct((n, d), jnp.float32),
    )(accp, degp, h, w_self, bias.reshape(1, d))


def _gidx_body(src_ref, typ_ref, out_ref, *, n_nodes):
    out_ref[...] = typ_ref[...] * n_nodes + src_ref[...]


def _gidx(src2, typ2, n_nodes):
    rows, k = src2.shape
    tr = NW * 2
    return pl.pallas_call(
        functools.partial(_gidx_body, n_nodes=n_nodes),
        grid=(rows // tr,),
        in_specs=[pl.BlockSpec((tr, k), lambda i: (i, 0)),
                  pl.BlockSpec((tr, k), lambda i: (i, 0))],
        out_specs=pl.BlockSpec((tr, k), lambda i: (i, 0)),
        out_shape=jax.ShapeDtypeStruct((rows, k), jnp.int32),
    )(src2, typ2)


HALF = 16


@functools.cache
def _make_aggregate(n_nodes, d, n_acc):
    mesh = plsc.VectorSubcoreMesh(core_axis_name="c", subcore_axis_name="s",
                                  num_cores=NC, num_subcores=NS)
    out_type = jax.ShapeDtypeStruct((NC, n_acc, d), jnp.float32)
    scratch = [
        pltpu.VMEM_SHARED((n_acc, d), jnp.float32),
        pltpu.VMEM((HALF, K), jnp.int32),
        pltpu.VMEM((HALF, K), jnp.int32),
        pltpu.VMEM((K, d), jnp.float32),
        pltpu.VMEM((K, d), jnp.float32),
        pltpu.SemaphoreType.DMA,
        pltpu.SemaphoreType.DMA,
    ]
    rows_per_tile = n_acc // NS

    def body(gidx2, dst2, table, accout, acc_sh, gidxv, dstv, rows0, rows1,
             sem0, sem1):
        c = lax.axis_index("c")
        s = lax.axis_index("s")

        zeros16 = jnp.zeros((LANES,), jnp.float32)

        def init_row(i, carry):
            for k in range(d // LANES):
                rows0[i, pl.ds(k * LANES, LANES)] = zeros16
            return carry
        lax.fori_loop(0, K, init_row, 0)

        for j in range(rows_per_tile // K):
            base = s * rows_per_tile + j * K
            pltpu.sync_copy(rows0, acc_sh.at[pl.ds(base, K)])
        plsc.subcore_barrier()

        def gather(j, buf, sem):
            return pltpu.async_copy(table.at[gidxv.at[j]], buf, sem)

        def gwait(j, buf, sem):
            pltpu.make_async_copy(table.at[gidxv.at[j]], buf, sem).wait()

        def scatter(j, buf):
            pltpu.sync_copy(buf, acc_sh.at[dstv.at[j]], add=True)

        def load_meta(base_row):
            pltpu.sync_copy(gidx2.at[pl.ds(base_row, HALF)], gidxv)
            pltpu.sync_copy(dst2.at[pl.ds(base_row, HALF)], dstv)

        def run_half(base_row):
            load_meta(base_row)
            gather(0, rows0, sem0)

            def it(i, carry):
                j = i * 2
                gather(j + 1, rows1, sem1)
                gwait(j, rows0, sem0)
                scatter(j, rows0)

                @pl.when(j + 2 < HALF)
                def _():
                    gather(j + 2, rows0, sem0)

                gwait(j + 1, rows1, sem1)
                scatter(j + 1, rows1)
                return carry
            lax.fori_loop(0, HALF // 2, it, 0)

        def run_half_phased(base_row):
            load_meta(base_row)

            def it(i, carry):
                j = i * 2
                gather(j, rows0, sem0)
                gather(j + 1, rows1, sem1)
                gwait(j, rows0, sem0)
                gwait(j + 1, rows1, sem1)
                scatter(j, rows0)
                scatter(j + 1, rows1)
                return carry
            lax.fori_loop(0, HALF // 2, it, 0)

        n_stage_fast = 9
        @pl.when(c == 0)
        def _():
            for h in range(n_stage_fast):
                run_half(s * (n_stage_fast * HALF) + h * HALF)

        @pl.when(c == 1)
        def _():
            run_half_phased(NS * n_stage_fast * HALF + s * HALF)

        plsc.subcore_barrier()
        for j in range(rows_per_tile // K):
            base = s * rows_per_tile + j * K
            pltpu.sync_copy(acc_sh.at[pl.ds(base, K)], accout.at[c, pl.ds(base, K)])

    return pl.kernel(body, out_type=out_type, mesh=mesh, scratch_types=scratch)


@functools.cache
def _make_deg(n_acc, d):
    mesh = plsc.VectorSubcoreMesh(core_axis_name="c", subcore_axis_name="s",
                                  num_cores=NC, num_subcores=NS)
    out_type = jax.ShapeDtypeStruct((NC, n_acc, d), jnp.float32)
    scratch = [
        pltpu.VMEM_SHARED((n_acc, d), jnp.float32),
        pltpu.VMEM((CH, K), jnp.int32),
        pltpu.VMEM((K, d), jnp.float32),
        pltpu.SemaphoreType.DMA,
    ]
    rows_per_tile = n_acc // NS

    def body(dst2, zrows, degout, deg_sh, dstv, onesv, sem):
        c = lax.axis_index("c")
        s = lax.axis_index("s")
        wid = c * NS + s
        ones16 = jnp.ones((LANES,), jnp.float32)

        def init_row(i, carry):
            for k in range(d // LANES):
                onesv[i, pl.ds(k * LANES, LANES)] = ones16
            return carry
        lax.fori_loop(0, K, init_row, 0)

        for j in range(rows_per_tile // K):
            base = s * rows_per_tile + j * K
            pltpu.sync_copy(zrows, deg_sh.at[pl.ds(base, K)])
        plsc.subcore_barrier()

        pltpu.sync_copy(dst2.at[pl.ds(wid * CH, CH)], dstv)

        def chunk(j, carry):
            pltpu.async_copy(onesv, deg_sh.at[dstv.at[j]], sem, add=True)
            return carry
        lax.fori_loop(0, CH, chunk, 0)

        def drain(j, carry):
            pltpu.make_async_copy(onesv, deg_sh.at[dstv.at[j]], sem).wait()
            return carry
        lax.fori_loop(0, CH, drain, 0)

        plsc.subcore_barrier()
        for j in range(rows_per_tile // K):
            base = s * rows_per_tile + j * K
            pltpu.sync_copy(deg_sh.at[pl.ds(base, K)], degout.at[c, pl.ds(base, K)])

    return pl.kernel(body, out_type=out_type, mesh=mesh, scratch_types=scratch)


@functools.cache
def _make_gather(n_q, d):
    mesh = plsc.VectorSubcoreMesh(core_axis_name="c", subcore_axis_name="s",
                                  num_cores=NC, num_subcores=NS)
    rows_pw = n_q // NW
    out_type = [jax.ShapeDtypeStruct((n_q, d), jnp.float32),
                jax.ShapeDtypeStruct((n_q, d), jnp.float32)]
    scratch = [
        pltpu.VMEM((rows_pw,), jnp.int32),
        pltpu.VMEM((rows_pw, d), jnp.float32),
        pltpu.SemaphoreType.DMA,
    ]

    def body(h1_hbm, h2_hbm, ids_hbm, o1, o2, idxv, rowsv, sem):
        c = lax.axis_index("c")
        s = lax.axis_index("s")
        base = (c * NS + s) * rows_pw
        pltpu.sync_copy(ids_hbm.at[pl.ds(base, rows_pw)], idxv)
        pltpu.async_copy(h1_hbm.at[idxv], rowsv, sem).wait()
        pltpu.sync_copy(rowsv, o1.at[pl.ds(base, rows_pw)])
        pltpu.async_copy(h2_hbm.at[idxv], rowsv, sem).wait()
        pltpu.sync_copy(rowsv, o2.at[pl.ds(base, rows_pw)])

    return pl.kernel(body, out_type=out_type, mesh=mesh, scratch_types=scratch)


def kernel(x, edge_index, edge_type, node_ids, W1_basis, W1_comp, W1_self, b1,
           W2_basis, W2_comp, W2_self, b2):
    n, d = x.shape
    e = edge_type.shape[0]
    n_q = node_ids.shape[0]
    n_acc = ((n + 1 + NS * K - 1) // (NS * K)) * (NS * K)

    e_pad = NW * EPW
    pad = e_pad - e
    src2 = jnp.concatenate([edge_index[0], jnp.zeros((pad,), jnp.int32)]).reshape(NW * CH, K)
    dst1 = jnp.concatenate([edge_index[1], jnp.full((pad,), n, jnp.int32)])
    dst2 = dst1.reshape(NW * CH, K)
    typ2 = jnp.concatenate([edge_type, jnp.zeros((pad,), jnp.int32)]).reshape(NW * CH, K)

    agg = _make_aggregate(n, d, n_acc)
    deg_k = _make_deg(n_acc, d)
    gather2 = _make_gather(n_q, d)

    zrows = jnp.zeros((K, d), jnp.float32)
    degp = _deg_reduce(deg_k(dst2, zrows))
    gidx2 = _gidx(src2, typ2, n)

    w1 = _mix(W1_comp, W1_basis)
    hrel1 = _project(x, w1).reshape(-1, d)
    accp1 = agg(gidx2, dst2, hrel1)
    h1 = _combine(accp1, degp, x, W1_self, b1)

    w2 = _mix(W2_comp, W2_basis)
    hrel2 = _project(h1, w2).reshape(-1, d)
    accp2 = agg(gidx2, dst2, hrel2)
    h2 = _combine(accp2, degp, h1, W2_self, b2)

    g1, g2 = gather2(h1, h2, node_ids)
    return jnp.concatenate([g1, g2], axis=-1)
